# Initial kernel scaffold; baseline (speedup 1.0000x reference)
import jax
import jax.numpy as jnp
from jax.experimental import pallas as pl


def kernel(node_feat, node_opcode, edge_index, node_config_feat, node_config_ids, config_edge_index, emb, ng0_Wl, ng0_bl, ng0_Wr, ng1_Wl, ng1_bl, ng1_Wr, ng2_Wl, ng2_bl, ng2_Wr, ng3_Wl, ng3_bl, ng3_Wr, cn0_Wl, cn0_bl, cn0_Wr, cn1_Wl, cn1_bl, cn1_Wr, cg0_Wl, cg0_bl, cg0_Wr, cg1_Wl, cg1_bl, cg1_Wr, cg2_Wl, cg2_bl, cg2_Wr, cg3_Wl, cg3_bl, cg3_Wr, prj_W, prj_b, d1_W, d2_W, d3_W):
    raise NotImplementedError("write your pallas kernel here")


# speedup vs baseline: 1.0000x; 1.0000x over previous
import functools

import jax
import jax.numpy as jnp
from jax.experimental import pallas as pl
from jax.experimental.pallas import tpu as pltpu

N = 50000; E = 800000; NC = 5000; C = 16; EC = 20000
NUM_OP = 120; FEAT = 140; CFG = 18; EMB = 32; D = 64


def _leaky(x):
    return jnp.where(x >= 0, x, 0.01 * x)


def _mean_agg(x, src, dst, n):
    s = jax.ops.segment_sum(x[src], dst, num_segments=n)
    deg = jax.ops.segment_sum(jnp.ones(src.shape[0], x.dtype), dst, num_segments=n)
    return s / jnp.maximum(deg, 1.0)[:, None]


def _sage(x, src, dst, n, Wl, bl, Wr):
    return _mean_agg(x, src, dst, n) @ Wl.T + bl + x @ Wr.T


def _head_body(pooled_ref, w1_ref, w2_ref, w3_ref, out_ref):
    h = _leaky(jnp.dot(pooled_ref[...], w1_ref[...].T, preferred_element_type=jnp.float32))
    h = _leaky(jnp.dot(h, w2_ref[...].T, preferred_element_type=jnp.float32))
    out_ref[...] = jnp.dot(h, w3_ref[...].T, preferred_element_type=jnp.float32)


def _head(pooled, d1_W, d2_W, d3_W):
    out = pl.pallas_call(
        _head_body,
        out_shape=jax.ShapeDtypeStruct((C, 1), jnp.float32),
    )(pooled, d1_W, d2_W, d3_W)
    return out.reshape(-1)


def kernel(node_feat, node_opcode, edge_index, node_config_feat, node_config_ids, config_edge_index, emb, ng0_Wl, ng0_bl, ng0_Wr, ng1_Wl, ng1_bl, ng1_Wr, ng2_Wl, ng2_bl, ng2_Wr, ng3_Wl, ng3_bl, ng3_Wr, cn0_Wl, cn0_bl, cn0_Wr, cn1_Wl, cn1_bl, cn1_Wr, cg0_Wl, cg0_bl, cg0_Wr, cg1_Wl, cg1_bl, cg1_Wr, cg2_Wl, cg2_bl, cg2_Wr, cg3_Wl, cg3_bl, cg3_Wr, prj_W, prj_b, d1_W, d2_W, d3_W):
    q = dict(locals())
    x = jnp.concatenate([node_feat, emb[node_opcode]], axis=1)
    src, dst = edge_index[0], edge_index[1]
    for i in range(4):
        x = _leaky(_sage(x, src, dst, N, q[f"ng{i}_Wl"], q[f"ng{i}_bl"], q[f"ng{i}_Wr"]))
    cn = _mean_agg(x, src, dst, N)[node_config_ids]
    csrc, cdst = config_edge_index[0], config_edge_index[1]
    for i in range(2):
        cn = _leaky(_sage(cn, csrc, cdst, NC, q[f"cn{i}_Wl"], q[f"cn{i}_bl"], q[f"cn{i}_Wr"]))
    xs = x[node_config_ids]
    pcf = _leaky(node_config_feat @ prj_W.T + prj_b)
    merged = jnp.concatenate([
        jnp.broadcast_to(cn[None], (C, NC, D)),
        jnp.broadcast_to(xs[None], (C, NC, D)),
        pcf,
    ], axis=-1)
    nrm = jnp.maximum(jnp.linalg.norm(merged, axis=-1, keepdims=True), 1e-12)
    merged = merged / nrm
    bx = merged.reshape(C * NC, 3 * D)
    off = (jnp.arange(C) * NC)[:, None]
    bsrc = (csrc[None, :] + off).reshape(-1)
    bdst = (cdst[None, :] + off).reshape(-1)
    for i in range(4):
        bx = _leaky(_sage(bx, bsrc, bdst, C * NC, q[f"cg{i}_Wl"], q[f"cg{i}_bl"], q[f"cg{i}_Wr"]))
    pooled = bx.reshape(C, NC, D).mean(axis=1)
    return _head(pooled, d1_W, d2_W, d3_W)


# speedup vs baseline: 2.2759x; 2.2759x over previous
import functools

import jax
import jax.numpy as jnp
from jax import lax
from jax.experimental import pallas as pl
from jax.experimental.pallas import tpu as pltpu
from jax.experimental.pallas import tpu_sc as plsc

N = 50000; E = 800000; NC = 5000; C = 16; EC = 20000
NUM_OP = 120; FEAT = 140; CFG = 18; EMB = 32; D = 64

NP = 50176
EP = 802816
ER = 6272
NCP = 5120
ECP = 20480
ECR = 160
BN = 512


def _leaky(x):
    return jnp.where(x >= 0, x, 0.01 * x)


_MESH = plsc.VectorSubcoreMesh(core_axis_name="c", subcore_axis_name="s")
_SC_PARAMS = pltpu.CompilerParams(use_tc_tiling_on_sc=False)


def _sc_segsum(np_nodes, rows_total, t):
    rpt = rows_total // 16
    slab = np_nodes // 16

    @functools.partial(
        pl.kernel,
        compiler_params=_SC_PARAMS,
        out_type=jax.ShapeDtypeStruct((4, np_nodes, 8), jnp.float32),
        mesh=_MESH,
        scratch_types=[
            pltpu.VMEM((rpt, 128), jnp.int32),
            pltpu.VMEM((rpt, 128), jnp.int32),
            pltpu.VMEM((128, 8), jnp.float32),
            pltpu.VMEM_SHARED((np_nodes, 8), jnp.float32),
            pltpu.SemaphoreType.DMA,
        ],
    )
    def k(y8, srcidx, dstp, zeros, out, src_t, dst_t, rows, acc, sem):
        c = lax.axis_index("c")
        s = lax.axis_index("s")
        pltpu.sync_copy(dstp.at[s], dst_t)
        for p in range(2):
            e = c * 4 + t * 2 + p
            pltpu.sync_copy(zeros.at[pl.ds(s * slab, slab)],
                            acc.at[pl.ds(s * slab, slab)])
            pltpu.sync_copy(srcidx.at[e, s], src_t)
            plsc.subcore_barrier()

            def body(j, carry):
                pltpu.async_copy(y8.at[src_t.at[j]], rows, sem).wait()
                pltpu.sync_copy(rows, acc.at[dst_t.at[j]], add=True)
                return carry

            lax.fori_loop(0, rpt, body, 0)
            plsc.subcore_barrier()
            pltpu.sync_copy(acc.at[pl.ds(s * slab, slab)],
                            out.at[c * 2 + p, pl.ds(s * slab, slab)])

    return k


def _sc_deg(np_nodes, half_rows):
    rpw = half_rows // 16
    slab = np_nodes // 16

    @functools.partial(
        pl.kernel,
        compiler_params=_SC_PARAMS,
        out_type=jax.ShapeDtypeStruct((2, np_nodes, 8), jnp.float32),
        mesh=_MESH,
        scratch_types=[
            pltpu.VMEM((rpw, 128), jnp.int32),
            pltpu.VMEM((128, 8), jnp.float32),
            pltpu.VMEM_SHARED((np_nodes, 8), jnp.float32),
        ],
    )
    def k(dst2, ones, zeros, out, dst_t, ones_v, acc):
        c = lax.axis_index("c")
        s = lax.axis_index("s")
        pltpu.sync_copy(zeros.at[pl.ds(s * slab, slab)], acc.at[pl.ds(s * slab, slab)])
        pltpu.sync_copy(dst2.at[c, s], dst_t)
        pltpu.sync_copy(ones, ones_v)
        plsc.subcore_barrier()

        def body(j, carry):
            pltpu.sync_copy(ones_v, acc.at[dst_t.at[j]], add=True)
            return carry

        lax.fori_loop(0, rpw, body, 0)
        plsc.subcore_barrier()
        pltpu.sync_copy(acc.at[pl.ds(s * slab, slab)], out.at[c, pl.ds(s * slab, slab)])

    return k


def _sc_gather():
    @functools.partial(
        pl.kernel,
        compiler_params=_SC_PARAMS,
        out_type=(
            jax.ShapeDtypeStruct((40, 128, 64), jnp.float32),
            jax.ShapeDtypeStruct((40, 128, 64), jnp.float32),
        ),
        mesh=_MESH,
        scratch_types=[
            pltpu.VMEM((1, 128), jnp.int32),
            pltpu.VMEM((128, 64), jnp.float32),
            pltpu.VMEM((128, 64), jnp.float32),
            pltpu.SemaphoreType.DMA,
            pltpu.SemaphoreType.DMA,
        ],
    )
    def k(x4, magg, ncidx, out_xs, out_cn, idx_t, rowa, rowb, sema, semb):
        c = lax.axis_index("c")
        s = lax.axis_index("s")
        w = s * 2 + c
        for t in range(2):
            j = w + 32 * t

            @pl.when(j < 40)
            def _():
                pltpu.sync_copy(ncidx.at[j], idx_t)
                ga = pltpu.async_copy(x4.at[idx_t.at[0]], rowa, sema)
                gb = pltpu.async_copy(magg.at[idx_t.at[0]], rowb, semb)
                ga.wait()
                pltpu.sync_copy(rowa, out_xs.at[j])
                gb.wait()
                pltpu.sync_copy(rowb, out_cn.at[j])

    return k


def _sc_cga(h):
    @functools.partial(
        pl.kernel,
        compiler_params=_SC_PARAMS,
        out_type=jax.ShapeDtypeStruct((8, NCP, 64), jnp.float32),
        mesh=_MESH,
        scratch_types=[
            pltpu.VMEM((10, 128), jnp.int32),
            pltpu.VMEM((10, 128), jnp.int32),
            pltpu.VMEM((128, 64), jnp.float32),
            pltpu.VMEM_SHARED((NCP, 64), jnp.float32),
            pltpu.SemaphoreType.DMA,
        ],
    )
    def k(bxf, bsrcp, cdstp, zerosb, out, src_t, dst_t, rows, acc, sem):
        c = lax.axis_index("c")
        s = lax.axis_index("s")
        pltpu.sync_copy(cdstp.at[s], dst_t)
        for i in range(4):
            b = c * 8 + h * 4 + i
            pltpu.sync_copy(zerosb.at[pl.ds(s * 320, 320)], acc.at[pl.ds(s * 320, 320)])
            pltpu.sync_copy(bsrcp.at[b, s], src_t)
            plsc.subcore_barrier()

            def body(j, carry):
                pltpu.async_copy(bxf.at[src_t.at[j]], rows, sem).wait()
                pltpu.sync_copy(rows, acc.at[dst_t.at[j]], add=True)
                return carry

            lax.fori_loop(0, 10, body, 0)
            plsc.subcore_barrier()
            pltpu.sync_copy(acc.at[pl.ds(s * 320, 320)],
                            out.at[c * 4 + i, pl.ds(s * 320, 320)])

    return k


def _full(shape):
    return pl.BlockSpec(shape, lambda *_: tuple(0 for _ in shape))


def _recip_of(dg_ref):
    return 1.0 / jnp.maximum(dg_ref[0, :, 0:1] + dg_ref[1, :, 0:1], 1.0)


def _cat8(a_ref, b_ref):
    return jnp.concatenate([a_ref[0], a_ref[1], b_ref[0], b_ref[1],
                            a_ref[2], a_ref[3], b_ref[2], b_ref[3]], axis=1)


def _tc_embw(emb, wcat):
    def body(e_ref, w_ref, o_ref):
        o_ref[...] = jnp.dot(e_ref[...], w_ref[...].T,
                             preferred_element_type=jnp.float32)

    return pl.pallas_call(
        body, out_shape=jax.ShapeDtypeStruct((NUM_OP, 2 * D), jnp.float32),
    )(emb, wcat)


def _tc_p0(nf, opc, embw, wnf, bl):
    def body(nf_ref, op_ref, ew_ref, w_ref, b_ref, yl_o, yr_o):
        y = jnp.dot(nf_ref[...], w_ref[...].T, preferred_element_type=jnp.float32)
        oh = (op_ref[...] == lax.broadcasted_iota(jnp.int32, (BN, NUM_OP), 1))
        y = y + jnp.dot(oh.astype(jnp.float32), ew_ref[...],
                        preferred_element_type=jnp.float32)
        yl_o[...] = y[:, :D]
        yr_o[...] = y[:, D:] + b_ref[...]

    return pl.pallas_call(
        body,
        grid=(NP // BN,),
        in_specs=[
            pl.BlockSpec((BN, FEAT), lambda i: (i, 0)),
            pl.BlockSpec((BN, 1), lambda i: (i, 0)),
            _full((NUM_OP, 2 * D)),
            _full((2 * D, FEAT)),
            _full((1, D)),
        ],
        out_specs=[pl.BlockSpec((BN, D), lambda i: (i, 0))] * 2,
        out_shape=[jax.ShapeDtypeStruct((NP, D), jnp.float32)] * 2,
    )(nf, opc, embw, wnf, bl)


def _tc_pmid(sa, sb, degs, yr, wl, wr, bl, np_nodes):
    def body(sa_ref, sb_ref, dg_ref, yr_ref, wl_ref, wr_ref, b_ref, yl_o, yr_o):
        m = _cat8(sa_ref, sb_ref) * _recip_of(dg_ref)
        x = _leaky(m + yr_ref[...])
        yl_o[...] = jnp.dot(x, wl_ref[...].T, preferred_element_type=jnp.float32)
        yr_o[...] = jnp.dot(x, wr_ref[...].T,
                            preferred_element_type=jnp.float32) + b_ref[...]

    return pl.pallas_call(
        body,
        grid=(np_nodes // BN,),
        in_specs=[
            pl.BlockSpec((4, BN, 8), lambda i: (0, i, 0)),
            pl.BlockSpec((4, BN, 8), lambda i: (0, i, 0)),
            pl.BlockSpec((2, BN, 8), lambda i: (0, i, 0)),
            pl.BlockSpec((BN, D), lambda i: (i, 0)),
            _full((D, D)), _full((D, D)), _full((1, D)),
        ],
        out_specs=[pl.BlockSpec((BN, D), lambda i: (i, 0))] * 2,
        out_shape=[jax.ShapeDtypeStruct((np_nodes, D), jnp.float32)] * 2,
    )(sa, sb, degs, yr, wl, wr, bl)


def _tc_pfin(sa, sb, degs, yr, np_nodes):
    def body(sa_ref, sb_ref, dg_ref, yr_ref, x_o):
        m = _cat8(sa_ref, sb_ref) * _recip_of(dg_ref)
        x_o[...] = _leaky(m + yr_ref[...])

    return pl.pallas_call(
        body,
        grid=(np_nodes // BN,),
        in_specs=[
            pl.BlockSpec((4, BN, 8), lambda i: (0, i, 0)),
            pl.BlockSpec((4, BN, 8), lambda i: (0, i, 0)),
            pl.BlockSpec((2, BN, 8), lambda i: (0, i, 0)),
            pl.BlockSpec((BN, D), lambda i: (i, 0)),
        ],
        out_specs=pl.BlockSpec((BN, D), lambda i: (i, 0)),
        out_shape=jax.ShapeDtypeStruct((np_nodes, D), jnp.float32),
    )(sa, sb, degs, yr)


def _tc_mean(sa, sb, degs, np_nodes):
    def body(sa_ref, sb_ref, dg_ref, m_o):
        m_o[...] = _cat8(sa_ref, sb_ref) * _recip_of(dg_ref)

    return pl.pallas_call(
        body,
        grid=(np_nodes // BN,),
        in_specs=[
            pl.BlockSpec((4, BN, 8), lambda i: (0, i, 0)),
            pl.BlockSpec((4, BN, 8), lambda i: (0, i, 0)),
            pl.BlockSpec((2, BN, 8), lambda i: (0, i, 0)),
        ],
        out_specs=pl.BlockSpec((BN, D), lambda i: (i, 0)),
        out_shape=jax.ShapeDtypeStruct((np_nodes, D), jnp.float32),
    )(sa, sb, degs)


def _tc_proj(x, wl, wr, bl, np_nodes):
    def body(x_ref, wl_ref, wr_ref, b_ref, yl_o, yr_o):
        yl_o[...] = jnp.dot(x_ref[...], wl_ref[...].T,
                            preferred_element_type=jnp.float32)
        yr_o[...] = jnp.dot(x_ref[...], wr_ref[...].T,
                            preferred_element_type=jnp.float32) + b_ref[...]

    return pl.pallas_call(
        body,
        grid=(np_nodes // BN,),
        in_specs=[
            pl.BlockSpec((BN, D), lambda i: (i, 0)),
            _full((D, D)), _full((D, D)), _full((1, D)),
        ],
        out_specs=[pl.BlockSpec((BN, D), lambda i: (i, 0))] * 2,
        out_shape=[jax.ShapeDtypeStruct((np_nodes, D), jnp.float32)] * 2,
    )(x, wl, wr, bl)


def _tc_pcf(ncf, pw, pb):
    def body(n_ref, w_ref, b_ref, o_ref):
        o_ref[...] = _leaky(
            jnp.dot(n_ref[0], w_ref[...].T,
                    preferred_element_type=jnp.float32) + b_ref[...])[None]

    return pl.pallas_call(
        body,
        grid=(C, NCP // BN),
        in_specs=[
            pl.BlockSpec((1, BN, CFG), lambda i, j: (i, j, 0)),
            _full((D, CFG)), _full((1, D)),
        ],
        out_specs=pl.BlockSpec((1, BN, D), lambda i, j: (i, j, 0)),
        out_shape=jax.ShapeDtypeStruct((C, NCP, D), jnp.float32),
    )(ncf, pw, pb)


def _tc_bp0(cn2, xs, pcf, wl, wr, bl):
    def body(cn_ref, xs_ref, p_ref, wl_ref, wr_ref, b_ref, yl_o, yr_o):
        cn = cn_ref[...]
        xs = xs_ref[...]
        p = p_ref[0]
        s2 = (jnp.sum(cn * cn, axis=1, keepdims=True)
              + jnp.sum(xs * xs, axis=1, keepdims=True)
              + jnp.sum(p * p, axis=1, keepdims=True))
        inv = 1.0 / jnp.maximum(jnp.sqrt(s2), 1e-12)
        wlt = wl_ref[...].T
        wrt = wr_ref[...].T
        u = (jnp.dot(cn, wlt[:D], preferred_element_type=jnp.float32)
             + jnp.dot(xs, wlt[D:2 * D], preferred_element_type=jnp.float32)
             + jnp.dot(p, wlt[2 * D:], preferred_element_type=jnp.float32))
        v = (jnp.dot(cn, wrt[:D], preferred_element_type=jnp.float32)
             + jnp.dot(xs, wrt[D:2 * D], preferred_element_type=jnp.float32)
             + jnp.dot(p, wrt[2 * D:], preferred_element_type=jnp.float32))
        yl_o[...] = (u * inv)[None]
        yr_o[...] = (v * inv + b_ref[...])[None]

    return pl.pallas_call(
        body,
        grid=(C, NCP // BN),
        in_specs=[
            pl.BlockSpec((BN, D), lambda i, j: (j, 0)),
            pl.BlockSpec((BN, D), lambda i, j: (j, 0)),
            pl.BlockSpec((1, BN, D), lambda i, j: (i, j, 0)),
            _full((D, 3 * D)), _full((D, 3 * D)), _full((1, D)),
        ],
        out_specs=[pl.BlockSpec((1, BN, D), lambda i, j: (i, j, 0))] * 2,
        out_shape=[jax.ShapeDtypeStruct((C, NCP, D), jnp.float32)] * 2,
    )(cn2, xs, pcf, wl, wr, bl)


def _tc_cgmid(s, degs, yrb, wl, wr, bl):
    def body(s_ref, dg_ref, yr_ref, wl_ref, wr_ref, b_ref, yl_o, yr_o):
        r = _recip_of(dg_ref)
        x = _leaky(s_ref[0] * r + yr_ref[0])
        yl_o[...] = jnp.dot(x, wl_ref[...].T,
                            preferred_element_type=jnp.float32)[None]
        yr_o[...] = (jnp.dot(x, wr_ref[...].T,
                             preferred_element_type=jnp.float32) + b_ref[...])[None]

    return pl.pallas_call(
        body,
        grid=(C, NCP // BN),
        in_specs=[
            pl.BlockSpec((1, BN, D), lambda i, j: (i, j, 0)),
            pl.BlockSpec((2, BN, 8), lambda i, j: (0, j, 0)),
            pl.BlockSpec((1, BN, D), lambda i, j: (i, j, 0)),
            _full((D, D)), _full((D, D)), _full((1, D)),
        ],
        out_specs=[pl.BlockSpec((1, BN, D), lambda i, j: (i, j, 0))] * 2,
        out_shape=[jax.ShapeDtypeStruct((C, NCP, D), jnp.float32)] * 2,
    )(s, degs, yrb, wl, wr, bl)


def _tc_cgpool(s, degs, yrb):
    def body(s_ref, dg_ref, yr_ref, o_ref):
        r = _recip_of(dg_ref)
        x = _leaky(s_ref[0] * r + yr_ref[0])
        mask = (lax.broadcasted_iota(jnp.int32, (NCP, 1), 0) < NC)
        o_ref[...] = (jnp.sum(jnp.where(mask, x, 0.0), axis=0,
                              keepdims=True) * (1.0 / NC))[None]

    return pl.pallas_call(
        body,
        grid=(C,),
        in_specs=[
            pl.BlockSpec((1, NCP, D), lambda i: (i, 0, 0)),
            pl.BlockSpec((2, NCP, 8), lambda i: (0, 0, 0)),
            pl.BlockSpec((1, NCP, D), lambda i: (i, 0, 0)),
        ],
        out_specs=pl.BlockSpec((1, 1, D), lambda i: (i, 0, 0)),
        out_shape=jax.ShapeDtypeStruct((C, 1, D), jnp.float32),
    )(s, degs, yrb)


def _tc_head(pooled, d1, d2, d3):
    def body(p_ref, w1_ref, w2_ref, w3_ref, o_ref):
        h = _leaky(jnp.dot(p_ref[...], w1_ref[...].T,
                           preferred_element_type=jnp.float32))
        h = _leaky(jnp.dot(h, w2_ref[...].T, preferred_element_type=jnp.float32))
        o_ref[...] = jnp.dot(h, w3_ref[...].T, preferred_element_type=jnp.float32)

    return pl.pallas_call(
        body, out_shape=jax.ShapeDtypeStruct((C, 1), jnp.float32),
    )(pooled, d1, d2, d3)


_seg_big = (_sc_segsum(NP, ER, 0), _sc_segsum(NP, ER, 1))
_seg_cfg = (_sc_segsum(NCP, ECR, 0), _sc_segsum(NCP, ECR, 1))
_deg_big = _sc_deg(NP, ER // 2)
_deg_cfg = _sc_deg(NCP, ECR // 2)
_gather_k = _sc_gather()
_cga_k = (_sc_cga(0), _sc_cga(1))


def kernel(node_feat, node_opcode, edge_index, node_config_feat, node_config_ids, config_edge_index, emb, ng0_Wl, ng0_bl, ng0_Wr, ng1_Wl, ng1_bl, ng1_Wr, ng2_Wl, ng2_bl, ng2_Wr, ng3_Wl, ng3_bl, ng3_Wr, cn0_Wl, cn0_bl, cn0_Wr, cn1_Wl, cn1_bl, cn1_Wr, cg0_Wl, cg0_bl, cg0_Wr, cg1_Wl, cg1_bl, cg1_Wr, cg2_Wl, cg2_bl, cg2_Wr, cg3_Wl, cg3_bl, cg3_Wr, prj_W, prj_b, d1_W, d2_W, d3_W):
    f32 = jnp.float32
    src = edge_index[0].astype(jnp.int32)
    dst = edge_index[1].astype(jnp.int32)
    csrc = config_edge_index[0].astype(jnp.int32)
    cdst = config_edge_index[1].astype(jnp.int32)
    ncids = node_config_ids.astype(jnp.int32)

    src_p = jnp.concatenate([src, jnp.zeros(EP - E, jnp.int32)])
    dst_p = jnp.concatenate([dst, jnp.full(EP - E, N, jnp.int32)])
    srcidx8 = jnp.stack([8 * src_p + i for i in range(8)]).reshape(8, 16, ER // 16, 128)
    dstp = dst_p.reshape(16, ER // 16, 128)
    dst2 = dst_p.reshape(2, 16, ER // 32, 128)
    csrc_p = jnp.concatenate([csrc, jnp.zeros(ECP - EC, jnp.int32)])
    cdst_p = jnp.concatenate([cdst, jnp.full(ECP - EC, NC, jnp.int32)])
    csrcidx8 = jnp.stack([8 * csrc_p + i for i in range(8)]).reshape(8, 16, ECR // 16, 128)
    cdstp = cdst_p.reshape(16, ECR // 16, 128)
    cdst2 = cdst_p.reshape(2, 16, ECR // 32, 128)
    ncidx = jnp.concatenate([ncids, jnp.zeros(NCP - NC, jnp.int32)]).reshape(40, 1, 128)
    bsrcp = (jnp.arange(C, dtype=jnp.int32)[:, None] * NCP
             + csrc_p[None, :]).reshape(C, 16, ECR // 16, 128)

    nf = jnp.pad(node_feat, ((0, NP - N), (0, 0)))
    opc = jnp.pad(node_opcode.astype(jnp.int32), (0, NP - N)).reshape(NP, 1)
    ncf = jnp.pad(node_config_feat, ((0, 0), (0, NCP - NC), (0, 0)))

    z8 = jnp.zeros((NP, 8), f32)
    z8c = jnp.zeros((NCP, 8), f32)
    zb = jnp.zeros((NCP, 64), f32)
    ones8 = jnp.ones((128, 8), f32)

    def agg_big(y):
        y8 = y.reshape(8 * NP, 8)
        return (_seg_big[0](y8, srcidx8, dstp, z8),
                _seg_big[1](y8, srcidx8, dstp, z8))

    def agg_cfg(y):
        y8 = y.reshape(8 * NCP, 8)
        return (_seg_cfg[0](y8, csrcidx8, cdstp, z8c),
                _seg_cfg[1](y8, csrcidx8, cdstp, z8c))

    def agg_b(ylb):
        bxf = ylb.reshape(C * NCP, D)
        c0 = _cga_k[0](bxf, bsrcp, cdstp, zb)
        c1 = _cga_k[1](bxf, bsrcp, cdstp, zb)
        return jnp.concatenate([c0[:4], c1[:4], c0[4:], c1[4:]], 0)

    degs = _deg_big(dst2, ones8, z8)
    degs_c = _deg_cfg(cdst2, ones8, z8c)

    wnf = jnp.concatenate([ng0_Wl[:, :FEAT], ng0_Wr[:, :FEAT]], 0)
    wcat = jnp.concatenate([ng0_Wl[:, FEAT:], ng0_Wr[:, FEAT:]], 0)
    embw = _tc_embw(emb, wcat)
    yl, yr = _tc_p0(nf, opc, embw, wnf, ng0_bl.reshape(1, D))
    q = dict(ng1=(ng1_Wl, ng1_Wr, ng1_bl), ng2=(ng2_Wl, ng2_Wr, ng2_bl),
             ng3=(ng3_Wl, ng3_Wr, ng3_bl))
    for name in ("ng1", "ng2", "ng3"):
        wl_i, wr_i, bl_i = q[name]
        sa, sb = agg_big(yl)
        yl, yr = _tc_pmid(sa, sb, degs, yr, wl_i, wr_i, bl_i.reshape(1, D), NP)
    sa, sb = agg_big(yl)
    x4 = _tc_pfin(sa, sb, degs, yr, NP)
    sa4, sb4 = agg_big(x4)
    magg = _tc_mean(sa4, sb4, degs, NP)

    xs_r, cn_r = _gather_k(x4, magg, ncidx)
    xs = xs_r.reshape(NCP, D)
    cn = cn_r.reshape(NCP, D)

    ylc, yrc = _tc_proj(cn, cn0_Wl, cn0_Wr, cn0_bl.reshape(1, D), NCP)
    sca, scb = agg_cfg(ylc)
    ylc, yrc = _tc_pmid(sca, scb, degs_c, yrc, cn1_Wl, cn1_Wr,
                        cn1_bl.reshape(1, D), NCP)
    sca, scb = agg_cfg(ylc)
    cn2 = _tc_pfin(sca, scb, degs_c, yrc, NCP)

    pcf = _tc_pcf(ncf, prj_W, prj_b.reshape(1, D))
    ylb, yrb = _tc_bp0(cn2, xs, pcf, cg0_Wl, cg0_Wr, cg0_bl.reshape(1, D))
    qb = dict(cg1=(cg1_Wl, cg1_Wr, cg1_bl), cg2=(cg2_Wl, cg2_Wr, cg2_bl),
              cg3=(cg3_Wl, cg3_Wr, cg3_bl))
    for name in ("cg1", "cg2", "cg3"):
        wl_i, wr_i, bl_i = qb[name]
        sbm = agg_b(ylb)
        ylb, yrb = _tc_cgmid(sbm, degs_c, yrb, wl_i, wr_i, bl_i.reshape(1, D))
    sbm = agg_b(ylb)
    pooled = _tc_cgpool(sbm, degs_c, yrb).reshape(C, D)
    return _tc_head(pooled, d1_W, d2_W, d3_W).reshape(-1)


# speedup vs baseline: 2.9343x; 1.2893x over previous
import functools

import jax
import jax.numpy as jnp
from jax import lax
from jax.experimental import pallas as pl
from jax.experimental.pallas import tpu as pltpu
from jax.experimental.pallas import tpu_sc as plsc

N = 50000; E = 800000; NC = 5000; C = 16; EC = 20000
NUM_OP = 120; FEAT = 140; CFG = 18; EMB = 32; D = 64

NP = 50176
EP = 802816
ER = 6272
NCP = 5120
ECP = 20480
ECR = 160
BN = 512


def _leaky(x):
    return jnp.where(x >= 0, x, 0.01 * x)


_MESH = plsc.VectorSubcoreMesh(core_axis_name="c", subcore_axis_name="s")
_SC_PARAMS = pltpu.CompilerParams(use_tc_tiling_on_sc=False)


def _sc_segsum(np_nodes, rows_total, t):
    rpt = rows_total // 16
    slab = np_nodes // 16

    @functools.partial(
        pl.kernel,
        compiler_params=_SC_PARAMS,
        out_type=jax.ShapeDtypeStruct((4, np_nodes, 8), jnp.float32),
        mesh=_MESH,
        scratch_types=[
            pltpu.VMEM((rpt, 128), jnp.int32),
            pltpu.VMEM((rpt, 128), jnp.int32),
            pltpu.VMEM((128, 8), jnp.float32),
            pltpu.VMEM((128, 8), jnp.float32),
            pltpu.VMEM((128, 8), jnp.float32),
            pltpu.VMEM((128, 8), jnp.float32),
            pltpu.VMEM_SHARED((np_nodes, 8), jnp.float32),
            pltpu.SemaphoreType.DMA,
            pltpu.SemaphoreType.DMA,
            pltpu.SemaphoreType.DMA,
            pltpu.SemaphoreType.DMA,
        ],
    )
    def k(y8, srcidx, dstp, zeros, out, src_t, dst_t,
          r0, r1, r2, r3, acc, m0, m1, m2, m3):
        bufs = (r0, r1, r2, r3)
        sems = (m0, m1, m2, m3)
        c = lax.axis_index("c")
        s = lax.axis_index("s")
        pltpu.sync_copy(dstp.at[s], dst_t)
        for p in range(2):
            e = c * 4 + t * 2 + p
            pltpu.sync_copy(zeros.at[pl.ds(s * slab, slab)],
                            acc.at[pl.ds(s * slab, slab)])
            pltpu.sync_copy(srcidx.at[e, s], src_t)
            plsc.subcore_barrier()

            def body(i, carry):
                j = i * 2
                descs = [pltpu.async_copy(y8.at[src_t.at[j + b]], bufs[b], sems[b])
                         for b in range(2)]
                for b in range(2):
                    descs[b].wait()
                    pltpu.sync_copy(bufs[b], acc.at[dst_t.at[j + b]], add=True)
                return carry

            lax.fori_loop(0, rpt // 2, body, 0)
            plsc.subcore_barrier()
            pltpu.sync_copy(acc.at[pl.ds(s * slab, slab)],
                            out.at[c * 2 + p, pl.ds(s * slab, slab)])

    return k


def _sc_deg(np_nodes, half_rows):
    rpw = half_rows // 16
    slab = np_nodes // 16

    @functools.partial(
        pl.kernel,
        compiler_params=_SC_PARAMS,
        out_type=jax.ShapeDtypeStruct((2, np_nodes, 8), jnp.float32),
        mesh=_MESH,
        scratch_types=[
            pltpu.VMEM((rpw, 128), jnp.int32),
            pltpu.VMEM((128, 8), jnp.float32),
            pltpu.VMEM_SHARED((np_nodes, 8), jnp.float32),
        ],
    )
    def k(dst2, ones, zeros, out, dst_t, ones_v, acc):
        c = lax.axis_index("c")
        s = lax.axis_index("s")
        pltpu.sync_copy(zeros.at[pl.ds(s * slab, slab)], acc.at[pl.ds(s * slab, slab)])
        pltpu.sync_copy(dst2.at[c, s], dst_t)
        pltpu.sync_copy(ones, ones_v)
        plsc.subcore_barrier()

        def body(j, carry):
            pltpu.sync_copy(ones_v, acc.at[dst_t.at[j]], add=True)
            return carry

        lax.fori_loop(0, rpw, body, 0)
        plsc.subcore_barrier()
        pltpu.sync_copy(acc.at[pl.ds(s * slab, slab)], out.at[c, pl.ds(s * slab, slab)])

    return k


def _sc_gather():
    @functools.partial(
        pl.kernel,
        compiler_params=_SC_PARAMS,
        out_type=(
            jax.ShapeDtypeStruct((40, 128, 64), jnp.float32),
            jax.ShapeDtypeStruct((40, 128, 64), jnp.float32),
        ),
        mesh=_MESH,
        scratch_types=[
            pltpu.VMEM((1, 128), jnp.int32),
            pltpu.VMEM((128, 64), jnp.float32),
            pltpu.VMEM((128, 64), jnp.float32),
            pltpu.SemaphoreType.DMA,
            pltpu.SemaphoreType.DMA,
        ],
    )
    def k(x4, magg, ncidx, out_xs, out_cn, idx_t, rowa, rowb, sema, semb):
        c = lax.axis_index("c")
        s = lax.axis_index("s")
        w = s * 2 + c
        for t in range(2):
            j = w + 32 * t

            @pl.when(j < 40)
            def _():
                pltpu.sync_copy(ncidx.at[j], idx_t)
                ga = pltpu.async_copy(x4.at[idx_t.at[0]], rowa, sema)
                gb = pltpu.async_copy(magg.at[idx_t.at[0]], rowb, semb)
                ga.wait()
                pltpu.sync_copy(rowa, out_xs.at[j])
                gb.wait()
                pltpu.sync_copy(rowb, out_cn.at[j])

    return k


def _sc_cga(h):
    @functools.partial(
        pl.kernel,
        compiler_params=_SC_PARAMS,
        out_type=jax.ShapeDtypeStruct((8, NCP, 64), jnp.float32),
        mesh=_MESH,
        scratch_types=[
            pltpu.VMEM((10, 128), jnp.int32),
            pltpu.VMEM((10, 128), jnp.int32),
            pltpu.VMEM((128, 64), jnp.float32),
            pltpu.VMEM_SHARED((NCP, 64), jnp.float32),
            pltpu.SemaphoreType.DMA,
        ],
    )
    def k(bxf, bsrcp, cdstp, zerosb, out, src_t, dst_t, rows, acc, sem):
        c = lax.axis_index("c")
        s = lax.axis_index("s")
        pltpu.sync_copy(cdstp.at[s], dst_t)
        for i in range(4):
            b = c * 8 + h * 4 + i
            pltpu.sync_copy(zerosb.at[pl.ds(s * 320, 320)], acc.at[pl.ds(s * 320, 320)])
            pltpu.sync_copy(bsrcp.at[b, s], src_t)
            plsc.subcore_barrier()

            def body(j, carry):
                pltpu.async_copy(bxf.at[src_t.at[j]], rows, sem).wait()
                pltpu.sync_copy(rows, acc.at[dst_t.at[j]], add=True)
                return carry

            lax.fori_loop(0, 10, body, 0)
            plsc.subcore_barrier()
            pltpu.sync_copy(acc.at[pl.ds(s * 320, 320)],
                            out.at[c * 4 + i, pl.ds(s * 320, 320)])

    return k


def _full(shape):
    return pl.BlockSpec(shape, lambda *_: tuple(0 for _ in shape))


def _recip_of(dg_ref):
    return 1.0 / jnp.maximum(dg_ref[0, :, 0:1] + dg_ref[1, :, 0:1], 1.0)


def _cat8(a_ref, b_ref):
    return jnp.concatenate([a_ref[0], a_ref[1], b_ref[0], b_ref[1],
                            a_ref[2], a_ref[3], b_ref[2], b_ref[3]], axis=1)


def _tc_embw(emb, wcat):
    def body(e_ref, w_ref, o_ref):
        o_ref[...] = jnp.dot(e_ref[...], w_ref[...].T,
                             preferred_element_type=jnp.float32, precision=lax.Precision.HIGHEST)

    return pl.pallas_call(
        body, out_shape=jax.ShapeDtypeStruct((NUM_OP, 2 * D), jnp.float32),
    )(emb, wcat)


def _tc_p0(nf, opc, embw, wnf, bl):
    def body(nf_ref, op_ref, ew_ref, w_ref, b_ref, yl_o, yr_o):
        y = jnp.dot(nf_ref[...], w_ref[...].T, preferred_element_type=jnp.float32, precision=lax.Precision.HIGHEST)
        oh = (op_ref[...] == lax.broadcasted_iota(jnp.int32, (BN, NUM_OP), 1))
        y = y + jnp.dot(oh.astype(jnp.float32), ew_ref[...],
                        preferred_element_type=jnp.float32, precision=lax.Precision.HIGHEST)
        yl_o[...] = y[:, :D]
        yr_o[...] = y[:, D:] + b_ref[...]

    return pl.pallas_call(
        body,
        grid=(NP // BN,),
        in_specs=[
            pl.BlockSpec((BN, FEAT), lambda i: (i, 0)),
            pl.BlockSpec((BN, 1), lambda i: (i, 0)),
            _full((NUM_OP, 2 * D)),
            _full((2 * D, FEAT)),
            _full((1, D)),
        ],
        out_specs=[pl.BlockSpec((BN, D), lambda i: (i, 0))] * 2,
        out_shape=[jax.ShapeDtypeStruct((NP, D), jnp.float32)] * 2,
    )(nf, opc, embw, wnf, bl)


def _tc_pmid(sa, sb, degs, yr, wl, wr, bl, np_nodes):
    def body(sa_ref, sb_ref, dg_ref, yr_ref, wl_ref, wr_ref, b_ref, yl_o, yr_o):
        m = _cat8(sa_ref, sb_ref) * _recip_of(dg_ref)
        x = _leaky(m + yr_ref[...])
        yl_o[...] = jnp.dot(x, wl_ref[...].T, preferred_element_type=jnp.float32, precision=lax.Precision.HIGHEST)
        yr_o[...] = jnp.dot(x, wr_ref[...].T,
                            preferred_element_type=jnp.float32, precision=lax.Precision.HIGHEST) + b_ref[...]

    return pl.pallas_call(
        body,
        grid=(np_nodes // BN,),
        in_specs=[
            pl.BlockSpec((4, BN, 8), lambda i: (0, i, 0)),
            pl.BlockSpec((4, BN, 8), lambda i: (0, i, 0)),
            pl.BlockSpec((2, BN, 8), lambda i: (0, i, 0)),
            pl.BlockSpec((BN, D), lambda i: (i, 0)),
            _full((D, D)), _full((D, D)), _full((1, D)),
        ],
        out_specs=[pl.BlockSpec((BN, D), lambda i: (i, 0))] * 2,
        out_shape=[jax.ShapeDtypeStruct((np_nodes, D), jnp.float32)] * 2,
    )(sa, sb, degs, yr, wl, wr, bl)


def _tc_pfin(sa, sb, degs, yr, np_nodes):
    def body(sa_ref, sb_ref, dg_ref, yr_ref, x_o):
        m = _cat8(sa_ref, sb_ref) * _recip_of(dg_ref)
        x_o[...] = _leaky(m + yr_ref[...])

    return pl.pallas_call(
        body,
        grid=(np_nodes // BN,),
        in_specs=[
            pl.BlockSpec((4, BN, 8), lambda i: (0, i, 0)),
            pl.BlockSpec((4, BN, 8), lambda i: (0, i, 0)),
            pl.BlockSpec((2, BN, 8), lambda i: (0, i, 0)),
            pl.BlockSpec((BN, D), lambda i: (i, 0)),
        ],
        out_specs=pl.BlockSpec((BN, D), lambda i: (i, 0)),
        out_shape=jax.ShapeDtypeStruct((np_nodes, D), jnp.float32),
    )(sa, sb, degs, yr)


def _tc_mean(sa, sb, degs, np_nodes):
    def body(sa_ref, sb_ref, dg_ref, m_o):
        m_o[...] = _cat8(sa_ref, sb_ref) * _recip_of(dg_ref)

    return pl.pallas_call(
        body,
        grid=(np_nodes // BN,),
        in_specs=[
            pl.BlockSpec((4, BN, 8), lambda i: (0, i, 0)),
            pl.BlockSpec((4, BN, 8), lambda i: (0, i, 0)),
            pl.BlockSpec((2, BN, 8), lambda i: (0, i, 0)),
        ],
        out_specs=pl.BlockSpec((BN, D), lambda i: (i, 0)),
        out_shape=jax.ShapeDtypeStruct((np_nodes, D), jnp.float32),
    )(sa, sb, degs)


def _tc_proj(x, wl, wr, bl, np_nodes):
    def body(x_ref, wl_ref, wr_ref, b_ref, yl_o, yr_o):
        yl_o[...] = jnp.dot(x_ref[...], wl_ref[...].T,
                            preferred_element_type=jnp.float32, precision=lax.Precision.HIGHEST)
        yr_o[...] = jnp.dot(x_ref[...], wr_ref[...].T,
                            preferred_element_type=jnp.float32, precision=lax.Precision.HIGHEST) + b_ref[...]

    return pl.pallas_call(
        body,
        grid=(np_nodes // BN,),
        in_specs=[
            pl.BlockSpec((BN, D), lambda i: (i, 0)),
            _full((D, D)), _full((D, D)), _full((1, D)),
        ],
        out_specs=[pl.BlockSpec((BN, D), lambda i: (i, 0))] * 2,
        out_shape=[jax.ShapeDtypeStruct((np_nodes, D), jnp.float32)] * 2,
    )(x, wl, wr, bl)


def _tc_pcf(ncf, pw, pb):
    def body(n_ref, w_ref, b_ref, o_ref):
        o_ref[...] = _leaky(
            jnp.dot(n_ref[0], w_ref[...].T,
                    preferred_element_type=jnp.float32, precision=lax.Precision.HIGHEST) + b_ref[...])[None]

    return pl.pallas_call(
        body,
        grid=(C, NCP // BN),
        in_specs=[
            pl.BlockSpec((1, BN, CFG), lambda i, j: (i, j, 0)),
            _full((D, CFG)), _full((1, D)),
        ],
        out_specs=pl.BlockSpec((1, BN, D), lambda i, j: (i, j, 0)),
        out_shape=jax.ShapeDtypeStruct((C, NCP, D), jnp.float32),
    )(ncf, pw, pb)


def _tc_bp0(cn2, xs, pcf, wl, wr, bl):
    def body(cn_ref, xs_ref, p_ref, wl_ref, wr_ref, b_ref, yl_o, yr_o):
        cn = cn_ref[...]
        xs = xs_ref[...]
        p = p_ref[0]
        s2 = (jnp.sum(cn * cn, axis=1, keepdims=True)
              + jnp.sum(xs * xs, axis=1, keepdims=True)
              + jnp.sum(p * p, axis=1, keepdims=True))
        inv = 1.0 / jnp.maximum(jnp.sqrt(s2), 1e-12)
        wlt = wl_ref[...].T
        wrt = wr_ref[...].T
        u = (jnp.dot(cn, wlt[:D], preferred_element_type=jnp.float32, precision=lax.Precision.HIGHEST)
             + jnp.dot(xs, wlt[D:2 * D], preferred_element_type=jnp.float32, precision=lax.Precision.HIGHEST)
             + jnp.dot(p, wlt[2 * D:], preferred_element_type=jnp.float32, precision=lax.Precision.HIGHEST))
        v = (jnp.dot(cn, wrt[:D], preferred_element_type=jnp.float32, precision=lax.Precision.HIGHEST)
             + jnp.dot(xs, wrt[D:2 * D], preferred_element_type=jnp.float32, precision=lax.Precision.HIGHEST)
             + jnp.dot(p, wrt[2 * D:], preferred_element_type=jnp.float32, precision=lax.Precision.HIGHEST))
        yl_o[...] = (u * inv)[None]
        yr_o[...] = (v * inv + b_ref[...])[None]

    return pl.pallas_call(
        body,
        grid=(C, NCP // BN),
        in_specs=[
            pl.BlockSpec((BN, D), lambda i, j: (j, 0)),
            pl.BlockSpec((BN, D), lambda i, j: (j, 0)),
            pl.BlockSpec((1, BN, D), lambda i, j: (i, j, 0)),
            _full((D, 3 * D)), _full((D, 3 * D)), _full((1, D)),
        ],
        out_specs=[pl.BlockSpec((1, BN, D), lambda i, j: (i, j, 0))] * 2,
        out_shape=[jax.ShapeDtypeStruct((C, NCP, D), jnp.float32)] * 2,
    )(cn2, xs, pcf, wl, wr, bl)


def _tc_cgmid(s, degs, yrb, wl, wr, bl):
    def body(s_ref, dg_ref, yr_ref, wl_ref, wr_ref, b_ref, yl_o, yr_o):
        r = _recip_of(dg_ref)
        x = _leaky(s_ref[0] * r + yr_ref[0])
        yl_o[...] = jnp.dot(x, wl_ref[...].T,
                            preferred_element_type=jnp.float32, precision=lax.Precision.HIGHEST)[None]
        yr_o[...] = (jnp.dot(x, wr_ref[...].T,
                             preferred_element_type=jnp.float32, precision=lax.Precision.HIGHEST) + b_ref[...])[None]

    return pl.pallas_call(
        body,
        grid=(C, NCP // BN),
        in_specs=[
            pl.BlockSpec((1, BN, D), lambda i, j: (i, j, 0)),
            pl.BlockSpec((2, BN, 8), lambda i, j: (0, j, 0)),
            pl.BlockSpec((1, BN, D), lambda i, j: (i, j, 0)),
            _full((D, D)), _full((D, D)), _full((1, D)),
        ],
        out_specs=[pl.BlockSpec((1, BN, D), lambda i, j: (i, j, 0))] * 2,
        out_shape=[jax.ShapeDtypeStruct((C, NCP, D), jnp.float32)] * 2,
    )(s, degs, yrb, wl, wr, bl)


def _tc_cgpool(s, degs, yrb):
    def body(s_ref, dg_ref, yr_ref, o_ref):
        r = _recip_of(dg_ref)
        x = _leaky(s_ref[0] * r + yr_ref[0])
        mask = (lax.broadcasted_iota(jnp.int32, (NCP, 1), 0) < NC)
        o_ref[...] = (jnp.sum(jnp.where(mask, x, 0.0), axis=0,
                              keepdims=True) * (1.0 / NC))[None]

    return pl.pallas_call(
        body,
        grid=(C,),
        in_specs=[
            pl.BlockSpec((1, NCP, D), lambda i: (i, 0, 0)),
            pl.BlockSpec((2, NCP, 8), lambda i: (0, 0, 0)),
            pl.BlockSpec((1, NCP, D), lambda i: (i, 0, 0)),
        ],
        out_specs=pl.BlockSpec((1, 1, D), lambda i: (i, 0, 0)),
        out_shape=jax.ShapeDtypeStruct((C, 1, D), jnp.float32),
    )(s, degs, yrb)


def _tc_head(pooled, d1, d2, d3):
    def body(p_ref, w1_ref, w2_ref, w3_ref, o_ref):
        h = _leaky(jnp.dot(p_ref[...], w1_ref[...].T,
                           preferred_element_type=jnp.float32, precision=lax.Precision.HIGHEST))
        h = _leaky(jnp.dot(h, w2_ref[...].T, preferred_element_type=jnp.float32, precision=lax.Precision.HIGHEST))
        o_ref[...] = jnp.dot(h, w3_ref[...].T, preferred_element_type=jnp.float32, precision=lax.Precision.HIGHEST)

    return pl.pallas_call(
        body, out_shape=jax.ShapeDtypeStruct((C, 1), jnp.float32),
    )(pooled, d1, d2, d3)


_seg_big = (_sc_segsum(NP, ER, 0), _sc_segsum(NP, ER, 1))
_seg_cfg = (_sc_segsum(NCP, ECR, 0), _sc_segsum(NCP, ECR, 1))
_deg_big = _sc_deg(NP, ER // 2)
_deg_cfg = _sc_deg(NCP, ECR // 2)
_gather_k = _sc_gather()
_cga_k = (_sc_cga(0), _sc_cga(1))


def kernel(node_feat, node_opcode, edge_index, node_config_feat, node_config_ids, config_edge_index, emb, ng0_Wl, ng0_bl, ng0_Wr, ng1_Wl, ng1_bl, ng1_Wr, ng2_Wl, ng2_bl, ng2_Wr, ng3_Wl, ng3_bl, ng3_Wr, cn0_Wl, cn0_bl, cn0_Wr, cn1_Wl, cn1_bl, cn1_Wr, cg0_Wl, cg0_bl, cg0_Wr, cg1_Wl, cg1_bl, cg1_Wr, cg2_Wl, cg2_bl, cg2_Wr, cg3_Wl, cg3_bl, cg3_Wr, prj_W, prj_b, d1_W, d2_W, d3_W):
    f32 = jnp.float32
    src = edge_index[0].astype(jnp.int32)
    dst = edge_index[1].astype(jnp.int32)
    csrc = config_edge_index[0].astype(jnp.int32)
    cdst = config_edge_index[1].astype(jnp.int32)
    ncids = node_config_ids.astype(jnp.int32)

    src_p = jnp.concatenate([src, jnp.zeros(EP - E, jnp.int32)])
    dst_p = jnp.concatenate([dst, jnp.full(EP - E, N, jnp.int32)])
    srcidx8 = jnp.stack([8 * src_p + i for i in range(8)]).reshape(8, 16, ER // 16, 128)
    dstp = dst_p.reshape(16, ER // 16, 128)
    dst2 = dst_p.reshape(2, 16, ER // 32, 128)
    csrc_p = jnp.concatenate([csrc, jnp.zeros(ECP - EC, jnp.int32)])
    cdst_p = jnp.concatenate([cdst, jnp.full(ECP - EC, NC, jnp.int32)])
    csrcidx8 = jnp.stack([8 * csrc_p + i for i in range(8)]).reshape(8, 16, ECR // 16, 128)
    cdstp = cdst_p.reshape(16, ECR // 16, 128)
    cdst2 = cdst_p.reshape(2, 16, ECR // 32, 128)
    ncidx = jnp.concatenate([ncids, jnp.zeros(NCP - NC, jnp.int32)]).reshape(40, 1, 128)
    bsrcp = (jnp.arange(C, dtype=jnp.int32)[:, None] * NCP
             + csrc_p[None, :]).reshape(C, 16, ECR // 16, 128)

    nf = jnp.pad(node_feat, ((0, NP - N), (0, 0)))
    opc = jnp.pad(node_opcode.astype(jnp.int32), (0, NP - N)).reshape(NP, 1)
    ncf = jnp.pad(node_config_feat, ((0, 0), (0, NCP - NC), (0, 0)))

    z8 = jnp.zeros((NP, 8), f32)
    z8c = jnp.zeros((NCP, 8), f32)
    zb = jnp.zeros((NCP, 64), f32)
    ones8 = jnp.ones((128, 8), f32)

    def agg_big(y):
        y8 = y.reshape(8 * NP, 8)
        return (_seg_big[0](y8, srcidx8, dstp, z8),
                _seg_big[1](y8, srcidx8, dstp, z8))

    def agg_cfg(y):
        y8 = y.reshape(8 * NCP, 8)
        return (_seg_cfg[0](y8, csrcidx8, cdstp, z8c),
                _seg_cfg[1](y8, csrcidx8, cdstp, z8c))

    def agg_b(ylb):
        bxf = ylb.reshape(C * NCP, D)
        c0 = _cga_k[0](bxf, bsrcp, cdstp, zb)
        c1 = _cga_k[1](bxf, bsrcp, cdstp, zb)
        return jnp.concatenate([c0[:4], c1[:4], c0[4:], c1[4:]], 0)

    degs = _deg_big(dst2, ones8, z8)
    degs_c = _deg_cfg(cdst2, ones8, z8c)

    wnf = jnp.concatenate([ng0_Wl[:, :FEAT], ng0_Wr[:, :FEAT]], 0)
    wcat = jnp.concatenate([ng0_Wl[:, FEAT:], ng0_Wr[:, FEAT:]], 0)
    embw = _tc_embw(emb, wcat)
    yl, yr = _tc_p0(nf, opc, embw, wnf, ng0_bl.reshape(1, D))
    q = dict(ng1=(ng1_Wl, ng1_Wr, ng1_bl), ng2=(ng2_Wl, ng2_Wr, ng2_bl),
             ng3=(ng3_Wl, ng3_Wr, ng3_bl))
    for name in ("ng1", "ng2", "ng3"):
        wl_i, wr_i, bl_i = q[name]
        sa, sb = agg_big(yl)
        yl, yr = _tc_pmid(sa, sb, degs, yr, wl_i, wr_i, bl_i.reshape(1, D), NP)
    sa, sb = agg_big(yl)
    x4 = _tc_pfin(sa, sb, degs, yr, NP)
    sa4, sb4 = agg_big(x4)
    magg = _tc_mean(sa4, sb4, degs, NP)

    xs_r, cn_r = _gather_k(x4, magg, ncidx)
    xs = xs_r.reshape(NCP, D)
    cn = cn_r.reshape(NCP, D)

    ylc, yrc = _tc_proj(cn, cn0_Wl, cn0_Wr, cn0_bl.reshape(1, D), NCP)
    sca, scb = agg_cfg(ylc)
    ylc, yrc = _tc_pmid(sca, scb, degs_c, yrc, cn1_Wl, cn1_Wr,
                        cn1_bl.reshape(1, D), NCP)
    sca, scb = agg_cfg(ylc)
    cn2 = _tc_pfin(sca, scb, degs_c, yrc, NCP)

    pcf = _tc_pcf(ncf, prj_W, prj_b.reshape(1, D))
    ylb, yrb = _tc_bp0(cn2, xs, pcf, cg0_Wl, cg0_Wr, cg0_bl.reshape(1, D))
    qb = dict(cg1=(cg1_Wl, cg1_Wr, cg1_bl), cg2=(cg2_Wl, cg2_Wr, cg2_bl),
              cg3=(cg3_Wl, cg3_Wr, cg3_bl))
    for name in ("cg1", "cg2", "cg3"):
        wl_i, wr_i, bl_i = qb[name]
        sbm = agg_b(ylb)
        ylb, yrb = _tc_cgmid(sbm, degs_c, yrb, wl_i, wr_i, bl_i.reshape(1, D))
    sbm = agg_b(ylb)
    pooled = _tc_cgpool(sbm, degs_c, yrb).reshape(C, D)
    return _tc_head(pooled, d1_W, d2_W, d3_W).reshape(-1)
